# baseline (speedup 1.0000x reference)
import jax
import jax.numpy as jnp
from jax import lax
from jax.experimental import pallas as pl
from jax.experimental.pallas import tpu as pltpu
from jax.experimental.pallas import tpu_sc as plsc

B = 4096
L = 50
DIM = 64
R = 1000001

NC = 2
NS = 16
NW = NC * NS
BPW = B // NW
NVEC = DIM // 16
NBUF = 8

CPB = 4096
NBLK = (R + CPB - 1) // CPB
R2 = NBLK * CPB // 2
LP = 56


def _pair_body(in_ref, out_ref):
  x = in_ref[...]
  xt = x.T
  out_ref[...] = jnp.concatenate(
      [xt[:CPB // 2], xt[CPB // 2:]], axis=1)


def _body(sent_hbm, tab2_hbm, out_hbm, idx_v, pidx_v, buf_v, out_v, *sems):
  wid = lax.axis_index("s") * NC + lax.axis_index("c")

  pltpu.sync_copy(sent_hbm.at[pl.ds(wid * BPW, BPW)], idx_v)

  G_OFF = (0, 16, 32, 34)

  def load_idx(c):
    return [idx_v[c, pl.ds(o, 16)] for o in G_OFF]

  def start(c, b):
    ivs = load_idx(c)
    zeros = jnp.zeros((16,), jnp.int32)
    pidx_v[b, pl.ds(40, 16)] = zeros
    for g, o in enumerate(G_OFF):
      iv = ivs[g]
      p = lax.shift_left(lax.shift_right_logical(iv, 12), 11) | (iv & 2047)
      pidx_v[b, pl.ds(o, 16)] = p
    pltpu.async_copy(tab2_hbm.at[pidx_v.at[b]], buf_v.at[b], sems[b])

  def wait(b):
    pltpu.make_async_copy(tab2_hbm.at[pidx_v.at[b]], buf_v.at[b],
                          sems[b]).wait()

  for b in range(NBUF):
    start(b, b)

  def compute(c, b):
    ivs = load_idx(c)
    hw = [(lax.shift_right_logical(iv, 11) & 1).astype(jnp.float32)
          for iv in ivs]
    dnums = jax.lax.GatherDimensionNumbers(
        offset_dims=(), collapsed_slice_dims=(0,), start_index_map=(0,))
    accs = [jnp.zeros((16,), jnp.float32) for _ in range(NVEC)]
    for l in range(L):
      if l < 48:
        g, lane = l // 16, l % 16
      else:
        g, lane = 3, l - 34
      w = jax.lax.gather(
          hw[g], jnp.full((16, 1), lane, jnp.int32), dnums, (1,),
          mode=jax.lax.GatherScatterMode.PROMISE_IN_BOUNDS)
      for j in range(NVEC):
        lo = buf_v[b, l, pl.ds(16 * j, 16)]
        hi = buf_v[b, l, pl.ds(64 + 16 * j, 16)]
        accs[j] = accs[j] + (lo + (hi - lo) * w)
    for j in range(NVEC):
      out_v[c, pl.ds(16 * j, 16)] = accs[j]

  def g_body(g, carry):
    for b in range(NBUF):
      c = NBUF * g + b
      wait(b)
      compute(c, b)

      @pl.when(g < BPW // NBUF - 1)
      def _():
        start(c + NBUF, b)
    return carry

  lax.fori_loop(0, BPW // NBUF, g_body, 0)

  pltpu.sync_copy(out_v, out_hbm.at[pl.ds(wid * BPW, BPW)])


@jax.jit
def _encode(sent, table):
  tab2 = pl.pallas_call(
      _pair_body,
      grid=(NBLK,),
      in_specs=[pl.BlockSpec((64, CPB), lambda c: (0, c))],
      out_specs=pl.BlockSpec((CPB // 2, 128), lambda c: (c, 0)),
      out_shape=jax.ShapeDtypeStruct((R2, 128), jnp.float32),
  )(table.T)

  mesh = plsc.VectorSubcoreMesh(core_axis_name="c", subcore_axis_name="s")
  return pl.kernel(
      _body,
      out_type=jax.ShapeDtypeStruct((B, DIM), jnp.float32),
      mesh=mesh,
      scratch_types=[
          pltpu.VMEM((BPW, L), jnp.int32),
          pltpu.VMEM((NBUF, LP), jnp.int32),
          pltpu.VMEM((NBUF, LP, 128), jnp.float32),
          pltpu.VMEM((BPW, DIM), jnp.float32),
      ] + [pltpu.SemaphoreType.DMA] * NBUF,
      compiler_params=pltpu.CompilerParams(use_tc_tiling_on_sc=True),
  )(sent, tab2)


def kernel(sent, table):
  return _encode(sent, table)


# speedup vs baseline: 3.3567x; 3.3567x over previous
import jax
import jax.numpy as jnp
from jax import lax
from jax.experimental import pallas as pl
from jax.experimental.pallas import tpu as pltpu
from jax.experimental.pallas import tpu_sc as plsc

B = 4096
L = 50
DIM = 64
R = 1000001

NC = 2
NS = 16
NW = NC * NS
BPW = B // NW
NVEC = DIM // 16
NBUF = 8

CPB = 4096
NBLK = (R + CPB - 1) // CPB
R2 = NBLK * CPB // 2


def _pair_body(in_ref, out_ref):
  x = in_ref[...]
  xt = x.T
  out_ref[...] = jnp.concatenate(
      [xt[:CPB // 2], xt[CPB // 2:]], axis=1)


def _body(sent_hbm, tab_hbm, out_hbm, idx_v, buf_v, out_v, *sems):
  wid = lax.axis_index("s") * NC + lax.axis_index("c")

  pltpu.sync_copy(sent_hbm.at[pl.ds(wid * BPW, BPW)], idx_v)

  def start(c, b):
    pltpu.async_copy(tab_hbm.at[idx_v.at[c]], buf_v.at[b], sems[b])

  def wait(c, b):
    pltpu.make_async_copy(tab_hbm.at[idx_v.at[c]], buf_v.at[b],
                          sems[b]).wait()

  for b in range(NBUF):
    start(b, b)

  def compute(c, b):
    accs = [buf_v[b, 0, pl.ds(16 * j, 16)] for j in range(NVEC)]
    for l in range(1, L):
      for j in range(NVEC):
        accs[j] = accs[j] + buf_v[b, l, pl.ds(16 * j, 16)]
    for j in range(NVEC):
      out_v[c, pl.ds(16 * j, 16)] = accs[j]

  def g_body(g, carry):
    for b in range(NBUF):
      c = NBUF * g + b
      wait(c, b)
      compute(c, b)

      @pl.when(g < BPW // NBUF - 1)
      def _():
        start(c + NBUF, b)
    return carry

  lax.fori_loop(0, BPW // NBUF, g_body, 0)

  pltpu.sync_copy(out_v, out_hbm.at[pl.ds(wid * BPW, BPW)])


@jax.jit
def _encode(sent, table):
  tab2 = pl.pallas_call(
      _pair_body,
      grid=(NBLK,),
      in_specs=[pl.BlockSpec((64, CPB), lambda c: (0, c))],
      out_specs=pl.BlockSpec((CPB // 2, 128), lambda c: (c, 0)),
      out_shape=jax.ShapeDtypeStruct((R2, 128), jnp.float32),
  )(table.T)

  sent_p = ((sent >> 12) << 12) | ((sent & 2047) << 1) | ((sent >> 11) & 1)

  mesh = plsc.VectorSubcoreMesh(core_axis_name="c", subcore_axis_name="s")
  return pl.kernel(
      _body,
      out_type=jax.ShapeDtypeStruct((B, DIM), jnp.float32),
      mesh=mesh,
      scratch_types=[
          pltpu.VMEM((BPW, L), jnp.int32),
          pltpu.VMEM((NBUF, L, DIM), jnp.float32),
          pltpu.VMEM((BPW, DIM), jnp.float32),
      ] + [pltpu.SemaphoreType.DMA] * NBUF,
      compiler_params=pltpu.CompilerParams(use_tc_tiling_on_sc=False),
  )(sent_p, tab2.reshape(2 * R2, DIM))


def kernel(sent, table):
  return _encode(sent, table)


# speedup vs baseline: 4.0276x; 1.1999x over previous
import jax
import jax.numpy as jnp
from jax import lax
from jax.experimental import pallas as pl
from jax.experimental.pallas import tpu as pltpu
from jax.experimental.pallas import tpu_sc as plsc

B = 4096
L = 50
DIM = 64
R = 1000001

NC = 2
NS = 16
NW = NC * NS
BPW = B // NW
NVEC = DIM // 16
NBUF = 8

CPB = 8192
SH = 13
NBLK = (R + CPB - 1) // CPB
R2 = NBLK * CPB // 2


def _pair_body(in_ref, out_ref):
  x = in_ref[...]
  xt = x.T
  out_ref[...] = jnp.concatenate(
      [xt[:CPB // 2], xt[CPB // 2:]], axis=1)


def _body(sent_hbm, tab_hbm, out_hbm, idx_v, buf_v, out_v, *sems):
  wid = lax.axis_index("s") * NC + lax.axis_index("c")

  pltpu.sync_copy(sent_hbm.at[pl.ds(wid * BPW, BPW)], idx_v)

  def start(c, b):
    pltpu.async_copy(tab_hbm.at[idx_v.at[c]], buf_v.at[b], sems[b])

  def wait(c, b):
    pltpu.make_async_copy(tab_hbm.at[idx_v.at[c]], buf_v.at[b],
                          sems[b]).wait()

  for b in range(NBUF):
    start(b, b)

  def compute(c, b):
    accs = [buf_v[b, 0, pl.ds(16 * j, 16)] for j in range(NVEC)]
    for l in range(1, L):
      for j in range(NVEC):
        accs[j] = accs[j] + buf_v[b, l, pl.ds(16 * j, 16)]
    for j in range(NVEC):
      out_v[c, pl.ds(16 * j, 16)] = accs[j]

  def g_body(g, carry):
    for b in range(NBUF):
      c = NBUF * g + b
      wait(c, b)
      compute(c, b)

      @pl.when(g < BPW // NBUF - 1)
      def _():
        start(c + NBUF, b)
    return carry

  lax.fori_loop(0, BPW // NBUF, g_body, 0)

  pltpu.sync_copy(out_v, out_hbm.at[pl.ds(wid * BPW, BPW)])


@jax.jit
def _encode(sent, table):
  tab2 = pl.pallas_call(
      _pair_body,
      grid=(NBLK,),
      in_specs=[pl.BlockSpec((64, CPB), lambda c: (0, c))],
      out_specs=pl.BlockSpec((CPB // 2, 128), lambda c: (c, 0)),
      out_shape=jax.ShapeDtypeStruct((R2, 128), jnp.float32),
  )(table.T)

  sent_p = (((sent >> SH) << SH)
            | ((sent & (CPB // 2 - 1)) << 1)
            | ((sent >> (SH - 1)) & 1))

  mesh = plsc.VectorSubcoreMesh(core_axis_name="c", subcore_axis_name="s")
  return pl.kernel(
      _body,
      out_type=jax.ShapeDtypeStruct((B, DIM), jnp.float32),
      mesh=mesh,
      scratch_types=[
          pltpu.VMEM((BPW, L), jnp.int32),
          pltpu.VMEM((NBUF, L, DIM), jnp.float32),
          pltpu.VMEM((BPW, DIM), jnp.float32),
      ] + [pltpu.SemaphoreType.DMA] * NBUF,
      compiler_params=pltpu.CompilerParams(use_tc_tiling_on_sc=False),
  )(sent_p, tab2.reshape(2 * R2, DIM))


def kernel(sent, table):
  return _encode(sent, table)
